# baseline (speedup 1.0000x reference)
import functools

import jax
import jax.numpy as jnp
from jax.experimental import pallas as pl
from jax.experimental.pallas import tpu as pltpu

K = 16
BIG = 3.0e38


def _knn_kernel(xn_ref, sq_ref, out_ref, *, n: int, d: int, chunk: int):
    xnv = xn_ref[0]
    sq = sq_ref[0]
    xbt = xnv.astype(jnp.bfloat16)
    xb = jnp.transpose(xbt)
    sq_row = jnp.transpose(sq)
    g_cnt = n // 128
    lane128 = jax.lax.broadcasted_iota(jnp.int32, (chunk, 128), 1)
    igs = [lane128 + g * 128 for g in range(g_cnt)]
    for c in range(n // chunk):
        xc = xb[c * chunk:(c + 1) * chunk]
        sc = sq[c * chunk:(c + 1) * chunk]
        p = jnp.dot(xc, xbt, preferred_element_type=jnp.float32)
        dist = (sc + (-2.0 * p)) + sq_row
        dsl = [dist[:, g * 128:(g + 1) * 128] for g in range(g_cnt)]
        for t in range(K):
            mval, midx = dsl[0], igs[0]
            for g in range(1, g_cnt):
                lt = dsl[g] < mval
                mval = jnp.where(lt, dsl[g], mval)
                midx = jnp.where(lt, igs[g], midx)
            m = jnp.min(mval, axis=1, keepdims=True)
            idx = jnp.min(jnp.where(mval == m, midx, n),
                          axis=1, keepdims=True)
            out_ref[0, c * chunk:(c + 1) * chunk, t] = idx[:, 0]
            eqlane = lane128 == jnp.bitwise_and(idx, 127)
            gi = jnp.right_shift(idx, 7)
            for g in range(g_cnt):
                dsl[g] = jnp.where(eqlane & (gi == g), BIG, dsl[g])


def kernel(x):
    b, d, n, _ = x.shape
    norm = jnp.sqrt(jnp.sum(x * x, axis=1, keepdims=True))
    xn = x / jnp.maximum(norm, 1e-12)
    xt = jnp.transpose(jnp.squeeze(xn, axis=-1), (0, 2, 1))
    x_square = jnp.sum(xt * xt, axis=-1, keepdims=True)
    xns = jnp.squeeze(xn, axis=-1)
    nn_idx = pl.pallas_call(
        functools.partial(_knn_kernel, n=n, d=d, chunk=256),
        grid=(b,),
        in_specs=[pl.BlockSpec((1, d, n), lambda bi: (bi, 0, 0)),
                  pl.BlockSpec((1, n, 1), lambda bi: (bi, 0, 0))],
        out_specs=pl.BlockSpec((1, n, K), lambda bi: (bi, 0, 0)),
        out_shape=jax.ShapeDtypeStruct((b, n, K), jnp.int32),
    )(xns, x_square)
    center_idx = jnp.broadcast_to(
        jnp.arange(n, dtype=jnp.int32)[None, :, None], (b, n, K))
    return jnp.stack((nn_idx, center_idx), axis=0)


# speedup vs baseline: 1.1672x; 1.1672x over previous
import functools

import jax
import jax.numpy as jnp
from jax.experimental import pallas as pl
from jax.experimental.pallas import tpu as pltpu

K = 16
BIG = 3.0e38


def _knn_kernel(xn_ref, sq_ref, out_ref, *, n: int, d: int, chunk: int):
    xnv = xn_ref[0]
    sq = sq_ref[0]
    xbt = xnv.astype(jnp.bfloat16)
    xb = jnp.transpose(xbt)
    sq_row = jnp.transpose(sq)
    lane = jax.lax.broadcasted_iota(jnp.int32, (chunk, n), 1)
    for c in range(n // chunk):
        xc = xb[c * chunk:(c + 1) * chunk]
        sc = sq[c * chunk:(c + 1) * chunk]
        p = jnp.dot(xc, xbt, preferred_element_type=jnp.float32)
        dist = (sc + (-2.0 * p)) + sq_row
        for t in range(K):
            m = jnp.min(dist, axis=1, keepdims=True)
            cand = jnp.where(dist == m, lane, n)
            idx = jnp.min(cand, axis=1, keepdims=True)
            out_ref[0, c * chunk:(c + 1) * chunk, t] = idx[:, 0]
            dist = jnp.where(lane == idx, BIG, dist)


def kernel(x):
    b, d, n, _ = x.shape
    norm = jnp.sqrt(jnp.sum(x * x, axis=1, keepdims=True))
    xn = x / jnp.maximum(norm, 1e-12)
    xt = jnp.transpose(jnp.squeeze(xn, axis=-1), (0, 2, 1))
    x_square = jnp.sum(xt * xt, axis=-1, keepdims=True)
    xns = jnp.squeeze(xn, axis=-1)
    nn_idx = pl.pallas_call(
        functools.partial(_knn_kernel, n=n, d=d, chunk=512),
        grid=(b,),
        in_specs=[pl.BlockSpec((1, d, n), lambda bi: (bi, 0, 0)),
                  pl.BlockSpec((1, n, 1), lambda bi: (bi, 0, 0))],
        out_specs=pl.BlockSpec((1, n, K), lambda bi: (bi, 0, 0)),
        out_shape=jax.ShapeDtypeStruct((b, n, K), jnp.int32),
    )(xns, x_square)
    center_idx = jnp.broadcast_to(
        jnp.arange(n, dtype=jnp.int32)[None, :, None], (b, n, K))
    return jnp.stack((nn_idx, center_idx), axis=0)


# speedup vs baseline: 1.1701x; 1.0025x over previous
import functools

import jax
import jax.numpy as jnp
from jax.experimental import pallas as pl
from jax.experimental.pallas import tpu as pltpu

K = 16
BIG = 3.0e38


def _knn_kernel(xn_ref, sq_ref, out_ref, *, n: int, d: int, chunk: int):
    xnv = xn_ref[0]
    sq = sq_ref[0]
    xbt = xnv.astype(jnp.bfloat16)
    xb = jnp.transpose(xbt)
    sq_row = jnp.transpose(sq)
    lane = jax.lax.broadcasted_iota(jnp.int32, (chunk, n), 1)
    for c in range(n // chunk):
        xc = xb[c * chunk:(c + 1) * chunk]
        sc = sq[c * chunk:(c + 1) * chunk]
        p = jnp.dot(xc, xbt, preferred_element_type=jnp.float32)
        dist = (sc + (-2.0 * p)) + sq_row
        for t in range(K):
            m = jnp.min(dist, axis=1, keepdims=True)
            cand = jnp.where(dist == m, lane, n)
            idx = jnp.min(cand, axis=1, keepdims=True)
            out_ref[0, c * chunk:(c + 1) * chunk, t] = idx[:, 0]
            dist = jnp.where(lane == idx, BIG, dist)


def kernel(x):
    b, d, n, _ = x.shape
    norm = jnp.sqrt(jnp.sum(x * x, axis=1, keepdims=True))
    xn = x / jnp.maximum(norm, 1e-12)
    xt = jnp.transpose(jnp.squeeze(xn, axis=-1), (0, 2, 1))
    x_square = jnp.sum(xt * xt, axis=-1, keepdims=True)
    xns = jnp.squeeze(xn, axis=-1)
    nn_idx = pl.pallas_call(
        functools.partial(_knn_kernel, n=n, d=d, chunk=1024),
        grid=(b,),
        in_specs=[pl.BlockSpec((1, d, n), lambda bi: (bi, 0, 0)),
                  pl.BlockSpec((1, n, 1), lambda bi: (bi, 0, 0))],
        out_specs=pl.BlockSpec((1, n, K), lambda bi: (bi, 0, 0)),
        out_shape=jax.ShapeDtypeStruct((b, n, K), jnp.int32),
    )(xns, x_square)
    center_idx = jnp.broadcast_to(
        jnp.arange(n, dtype=jnp.int32)[None, :, None], (b, n, K))
    return jnp.stack((nn_idx, center_idx), axis=0)
